# baseline (speedup 1.0000x reference)
import functools

import jax
import jax.numpy as jnp
import numpy as np
from jax.experimental import pallas as pl
from jax.experimental.pallas import tpu as pltpu
from jax.experimental.pallas import tpu_sc as plsc

S, D, H, E, K, F = 2048, 1024, 16, 8, 2, 2048
DH = D // H
BT = 256
BQ = 256
BK = 256
TM = 256
PAD_P = K * S + E * TM
NT = PAD_P // TM
BF = 512
RL = 128
NEG = -1e30
EPS = 1e-5


def _row_sum(x):
    w = x.shape[1]
    acc = x[:, 0:128]
    for c in range(1, w // 128):
        acc = acc + x[:, c * 128:(c + 1) * 128]
    a = acc[:, 0:8]
    for m in range(1, 16):
        a = a + acc[:, m * 8:(m + 1) * 8]
    a = a[:, 0:4] + a[:, 4:8]
    a = a[:, 0:2] + a[:, 2:4]
    return a[:, 0:1] + a[:, 1:2]


def _ln(x, g, b):
    mu = _row_sum(x) / x.shape[1]
    xc = x - mu
    var = _row_sum(xc * xc) / x.shape[1]
    return xc / jnp.sqrt(var + EPS) * g + b


def _ln_qkv_body(x_ref, g_ref, b_ref, wq_ref, bq_ref, wk_ref, bk_ref,
                 wv_ref, bv_ref, q_ref, k_ref, v_ref):
    h = _ln(x_ref[...], g_ref[...], b_ref[...])
    q_ref[...] = jnp.dot(h, wq_ref[...], preferred_element_type=jnp.float32) + bq_ref[...]
    k_ref[...] = jnp.dot(h, wk_ref[...], preferred_element_type=jnp.float32) + bk_ref[...]
    v_ref[...] = jnp.dot(h, wv_ref[...], preferred_element_type=jnp.float32) + bv_ref[...]


def _ln_qkv(xs, g, b, Wq, bq, Wk, bk, Wv, bv):
    row = pl.BlockSpec((BT, D), lambda i: (i, 0))
    full = pl.BlockSpec((D, D), lambda i: (0, 0))
    vec = pl.BlockSpec((1, D), lambda i: (0, 0))
    return pl.pallas_call(
        _ln_qkv_body,
        grid=(S // BT,),
        in_specs=[row, vec, vec, full, vec, full, vec, full, vec],
        out_specs=[row, row, row],
        out_shape=[jax.ShapeDtypeStruct((S, D), jnp.float32)] * 3,
    )(xs, g, b, Wq, bq, Wk, bk, Wv, bv)


def _attn_body(q_ref, k_ref, v_ref, o_ref):
    i = pl.program_id(1)
    q = q_ref[0]
    s = jax.lax.dot_general(q, k_ref[0], (((1,), (1,)), ((), ())),
                            preferred_element_type=jnp.float32) / np.float32(np.sqrt(DH))
    rows = i * BQ + jax.lax.broadcasted_iota(jnp.int32, (BQ, S), 0)
    cols = jax.lax.broadcasted_iota(jnp.int32, (BQ, S), 1)
    s = jnp.where(rows >= cols, s, -jnp.inf)
    m = jnp.max(s, axis=1, keepdims=True)
    p = jnp.exp(s - m)
    p = p / _row_sum(p)
    o_ref[0] = jnp.dot(p, v_ref[0], preferred_element_type=jnp.float32)


def _attn(q, k, v):
    qspec = pl.BlockSpec((1, BQ, DH), lambda h, i: (h, i, 0))
    kvspec = pl.BlockSpec((1, S, DH), lambda h, i: (h, 0, 0))
    return pl.pallas_call(
        _attn_body,
        grid=(H, S // BQ),
        in_specs=[qspec, kvspec, kvspec],
        out_specs=qspec,
        out_shape=jax.ShapeDtypeStruct((H, S, DH), jnp.float32),
    )(q, k, v)


def _proj_router_body(attn_ref, x_ref, wo_ref, bo_ref, g_ref, b_ref,
                      wr_ref, br_ref, xm_ref, h2_ref, probs_ref, idx_ref, w_ref):
    xm = x_ref[...] + (jnp.dot(attn_ref[...], wo_ref[...],
                               preferred_element_type=jnp.float32) + bo_ref[...])
    xm_ref[...] = xm
    h2 = _ln(xm, g_ref[...], b_ref[...])
    h2_ref[...] = h2
    logit = jnp.dot(h2, wr_ref[...], preferred_element_type=jnp.float32) + br_ref[...]
    lane = jax.lax.broadcasted_iota(jnp.int32, logit.shape, 1)
    m1 = jnp.max(logit, axis=1, keepdims=True)
    i1 = jnp.min(jnp.where(logit == m1, lane, RL - 1), axis=1, keepdims=True)
    l2 = jnp.where(lane == i1, NEG, logit)
    m2 = jnp.max(l2, axis=1, keepdims=True)
    i2 = jnp.min(jnp.where(l2 == m2, lane, RL - 1), axis=1, keepdims=True)
    pe = jnp.exp(logit - m1)
    probs_ref[...] = pe / _row_sum(pe)
    e2 = jnp.exp(m2 - m1)
    denom = 1.0 + e2
    w1 = 1.0 / denom
    w2 = e2 / denom
    idx_ref[...] = jnp.where(lane == 0, i1, i2)
    w_ref[...] = jnp.where(lane == 0, w1, w2)


def _proj_router(attn, xs, Wo, bo, g, b, Wr_pad, br_pad):
    row = pl.BlockSpec((BT, D), lambda i: (i, 0))
    rrow = pl.BlockSpec((BT, RL), lambda i: (i, 0))
    return pl.pallas_call(
        _proj_router_body,
        grid=(S // BT,),
        in_specs=[row, row,
                  pl.BlockSpec((D, D), lambda i: (0, 0)),
                  pl.BlockSpec((1, D), lambda i: (0, 0)),
                  pl.BlockSpec((1, D), lambda i: (0, 0)),
                  pl.BlockSpec((1, D), lambda i: (0, 0)),
                  pl.BlockSpec((D, RL), lambda i: (0, 0)),
                  pl.BlockSpec((1, RL), lambda i: (0, 0))],
        out_specs=[row, row, rrow, rrow, rrow],
        out_shape=[jax.ShapeDtypeStruct((S, D), jnp.float32),
                   jax.ShapeDtypeStruct((S, D), jnp.float32),
                   jax.ShapeDtypeStruct((S, RL), jnp.float32),
                   jax.ShapeDtypeStruct((S, RL), jnp.int32),
                   jax.ShapeDtypeStruct((S, RL), jnp.float32)],
    )(attn, xs, Wo, bo, g, b, Wr_pad, br_pad)


def _sc_gather(data, idx):
    N, Dd = data.shape
    M = idx.shape[0]
    SUB = 8
    CH = Dd // SUB
    W = 128
    data8 = data.reshape(N * SUB, CH)
    idx8 = (idx[:, None] * SUB + jnp.arange(SUB, dtype=jnp.int32)[None, :]
            ).reshape(1, M * SUB)
    mesh = plsc.VectorSubcoreMesh(core_axis_name="c", subcore_axis_name="s")

    @functools.partial(
        pl.kernel,
        out_type=jax.ShapeDtypeStruct((M * SUB, CH), data.dtype),
        mesh=mesh)
    def gk(x_hbm, i_hbm, o_hbm):
        def body(i_vmem, o_vmem):
            pltpu.sync_copy(x_hbm.at[i_vmem.at[0]], o_vmem)

        pltpu.emit_pipeline(
            body,
            grid=(M * SUB // W,),
            in_specs=[pl.BlockSpec((1, W), index_map=lambda i: (0, i))],
            out_specs=[pl.BlockSpec((W, CH), index_map=lambda i: (i, 0))],
            core_axis_name=("c", "s"),
            dimension_semantics=(pltpu.PARALLEL,),
        )(i_hbm, o_hbm)

    return gk(data8, idx8).reshape(M, Dd)


def _gate_up_body(te_ref, hs_ref, wg_ref, bg_ref, wu_ref, bu_ref, a_ref):
    hs = hs_ref[...]
    g = jnp.dot(hs, wg_ref[0], preferred_element_type=jnp.float32) + bg_ref[0]
    u = jnp.dot(hs, wu_ref[0], preferred_element_type=jnp.float32) + bu_ref[0]
    a_ref[...] = g * jax.lax.logistic(g) * u


def _gate_up(hs, Wg, bg3, Wu, bu3, te):
    grid_spec = pltpu.PrefetchScalarGridSpec(
        num_scalar_prefetch=1,
        grid=(F // BF, NT),
        in_specs=[
            pl.BlockSpec((TM, D), lambda f, t, te: (t, 0)),
            pl.BlockSpec((1, D, BF), lambda f, t, te: (te[t], 0, f)),
            pl.BlockSpec((1, 1, BF), lambda f, t, te: (te[t], 0, f)),
            pl.BlockSpec((1, D, BF), lambda f, t, te: (te[t], 0, f)),
            pl.BlockSpec((1, 1, BF), lambda f, t, te: (te[t], 0, f)),
        ],
        out_specs=pl.BlockSpec((TM, BF), lambda f, t, te: (t, f)),
    )
    return pl.pallas_call(
        _gate_up_body,
        grid_spec=grid_spec,
        out_shape=jax.ShapeDtypeStruct((PAD_P, F), jnp.float32),
    )(te, hs, Wg, bg3, Wu, bu3)


def _down_body(te_ref, a_ref, wd_ref, bd_ref, eo_ref):
    eo_ref[...] = jnp.dot(a_ref[...], wd_ref[0],
                          preferred_element_type=jnp.float32) + bd_ref[0]


def _down(a, Wd, bd3, te):
    grid_spec = pltpu.PrefetchScalarGridSpec(
        num_scalar_prefetch=1,
        grid=(NT,),
        in_specs=[
            pl.BlockSpec((TM, F), lambda t, te: (t, 0)),
            pl.BlockSpec((1, F, D), lambda t, te: (te[t], 0, 0)),
            pl.BlockSpec((1, 1, D), lambda t, te: (te[t], 0, 0)),
        ],
        out_specs=pl.BlockSpec((TM, D), lambda t, te: (t, 0)),
    )
    return pl.pallas_call(
        _down_body,
        grid_spec=grid_spec,
        out_shape=jax.ShapeDtypeStruct((PAD_P, D), jnp.float32),
    )(te, a, Wd, bd3)


def _combine_body(xm_ref, r2_ref, w_ref, o_ref):
    w1 = w_ref[:, 0:1]
    w2 = w_ref[:, 1:2]
    o_ref[...] = xm_ref[...] + w1 * r2_ref[:, :D] + w2 * r2_ref[:, D:]


def _combine(xm, r2, w):
    return pl.pallas_call(
        _combine_body,
        grid=(S // BT,),
        in_specs=[pl.BlockSpec((BT, D), lambda i: (i, 0)),
                  pl.BlockSpec((BT, 2 * D), lambda i: (i, 0)),
                  pl.BlockSpec((BT, RL), lambda i: (i, 0))],
        out_specs=pl.BlockSpec((BT, D), lambda i: (i, 0)),
        out_shape=jax.ShapeDtypeStruct((S, D), jnp.float32),
    )(xm, r2, w)


def _routing(idx2):
    ee = idx2.reshape(-1).astype(jnp.int32)
    order = jnp.argsort(ee, stable=True).astype(jnp.int32)
    counts = jnp.sum((ee[:, None] == jnp.arange(E)[None, :]).astype(jnp.int32), axis=0)
    padded = ((counts + TM - 1) // TM) * TM
    cum_pad = jnp.cumsum(padded)
    poff = cum_pad - padded
    cum_cnt = jnp.cumsum(counts)
    uoff = cum_cnt - counts
    e_sorted = ee[order]
    j = jnp.arange(K * S, dtype=jnp.int32)
    pos_sorted = (poff[e_sorted] + (j - uoff[e_sorted])).astype(jnp.int32)
    pos = jnp.zeros(K * S, jnp.int32).at[order].set(pos_sorted)
    src = jnp.zeros(PAD_P, jnp.int32).at[pos_sorted].set(order // K)
    te = jnp.searchsorted(cum_pad, jnp.arange(NT, dtype=jnp.int32) * TM,
                          side='right').astype(jnp.int32)
    te = jnp.minimum(te, E - 1)
    return pos, src, te


def kernel(x, ln1_g, ln1_b, ln2_g, ln2_b, Wq, bq, Wk, bk, Wv, bv, Wo, bo,
           Wr, br, Wg, bg, Wu, bu, Wd, bd):
    xs = x[0]
    g1 = ln1_g.reshape(1, D)
    b1 = ln1_b.reshape(1, D)
    g2 = ln2_g.reshape(1, D)
    b2 = ln2_b.reshape(1, D)
    Wr_pad = jnp.pad(Wr, ((0, 0), (0, RL - E)))
    br_pad = jnp.pad(br, (0, RL - E), constant_values=NEG).reshape(1, RL)

    q, k, v = _ln_qkv(xs, g1, b1, Wq, bq.reshape(1, D), Wk, bk.reshape(1, D),
                      Wv, bv.reshape(1, D))
    q = q.reshape(S, H, DH).transpose(1, 0, 2)
    k = k.reshape(S, H, DH).transpose(1, 0, 2)
    v = v.reshape(S, H, DH).transpose(1, 0, 2)
    attn = _attn(q, k, v).transpose(1, 0, 2).reshape(S, D)
    xm, h2, probs_pad, idx_pad, w_pad = _proj_router(
        attn, xs, Wo, bo.reshape(1, D), g2, b2, Wr_pad, br_pad)

    idx2 = idx_pad[:, :K]
    pos, src, te = _routing(idx2)

    hs = _sc_gather(h2, src)
    a = _gate_up(hs, Wg, bg.reshape(E, 1, F), Wu, bu.reshape(E, 1, F), te)
    eo = _down(a, Wd, bd.reshape(E, 1, D), te)
    r = _sc_gather(eo, pos)
    out = _combine(xm, r.reshape(S, K * D), w_pad)

    return (out[None], probs_pad[:, :E][None], idx2[None])


# speedup vs baseline: 1.1205x; 1.1205x over previous
import functools

import jax
import jax.numpy as jnp
import numpy as np
from jax.experimental import pallas as pl
from jax.experimental.pallas import tpu as pltpu
from jax.experimental.pallas import tpu_sc as plsc

S, D, H, E, K, F = 2048, 1024, 16, 8, 2, 2048
DH = D // H
BT = 256
BQ = 256
BK = 256
TM = 256
PAD_P = K * S + E * TM
NT = PAD_P // TM
BF = 512
RL = 128
NEG = -1e30
EPS = 1e-5


def _row_sum(x):
    w = x.shape[1]
    acc = x[:, 0:128]
    for c in range(1, w // 128):
        acc = acc + x[:, c * 128:(c + 1) * 128]
    a = acc[:, 0:8]
    for m in range(1, 16):
        a = a + acc[:, m * 8:(m + 1) * 8]
    a = a[:, 0:4] + a[:, 4:8]
    a = a[:, 0:2] + a[:, 2:4]
    return a[:, 0:1] + a[:, 1:2]


def _ln(x, g, b):
    mu = _row_sum(x) / x.shape[1]
    xc = x - mu
    var = _row_sum(xc * xc) / x.shape[1]
    return xc / jnp.sqrt(var + EPS) * g + b


def _ln_qkv_body(x_ref, g_ref, b_ref, wq_ref, bq_ref, wk_ref, bk_ref,
                 wv_ref, bv_ref, q_ref, k_ref, v_ref):
    h = _ln(x_ref[...], g_ref[...], b_ref[...])
    q_ref[...] = jnp.dot(h, wq_ref[...], preferred_element_type=jnp.float32) + bq_ref[...]
    k_ref[...] = jnp.dot(h, wk_ref[...], preferred_element_type=jnp.float32) + bk_ref[...]
    v_ref[...] = jnp.dot(h, wv_ref[...], preferred_element_type=jnp.float32) + bv_ref[...]


def _ln_qkv(xs, g, b, Wq, bq, Wk, bk, Wv, bv):
    row = pl.BlockSpec((BT, D), lambda i: (i, 0))
    full = pl.BlockSpec((D, D), lambda i: (0, 0))
    vec = pl.BlockSpec((1, D), lambda i: (0, 0))
    return pl.pallas_call(
        _ln_qkv_body,
        grid=(S // BT,),
        in_specs=[row, vec, vec, full, vec, full, vec, full, vec],
        out_specs=[row, row, row],
        out_shape=[jax.ShapeDtypeStruct((S, D), jnp.float32)] * 3,
    )(xs, g, b, Wq, bq, Wk, bk, Wv, bv)


def _attn_body(q_ref, k_ref, v_ref, o_ref):
    i = pl.program_id(1)
    q = q_ref[0]
    s = jax.lax.dot_general(q, k_ref[0], (((1,), (1,)), ((), ())),
                            preferred_element_type=jnp.float32) / np.float32(np.sqrt(DH))
    rows = i * BQ + jax.lax.broadcasted_iota(jnp.int32, (BQ, S), 0)
    cols = jax.lax.broadcasted_iota(jnp.int32, (BQ, S), 1)
    s = jnp.where(rows >= cols, s, -jnp.inf)
    m = jnp.max(s, axis=1, keepdims=True)
    u = jnp.exp(s - m)
    l = jnp.sum(u, axis=1, keepdims=True)
    o_ref[0] = jnp.dot(u, v_ref[0], preferred_element_type=jnp.float32) / l


def _attn(q, k, v):
    qspec = pl.BlockSpec((1, BQ, DH), lambda h, i: (h, i, 0))
    kvspec = pl.BlockSpec((1, S, DH), lambda h, i: (h, 0, 0))
    return pl.pallas_call(
        _attn_body,
        grid=(H, S // BQ),
        in_specs=[qspec, kvspec, kvspec],
        out_specs=qspec,
        out_shape=jax.ShapeDtypeStruct((H, S, DH), jnp.float32),
    )(q, k, v)


def _proj_router_body(attn_ref, x_ref, wo_ref, bo_ref, g_ref, b_ref,
                      wr_ref, br_ref, xm_ref, h2_ref, probs_ref, idx_ref, w_ref):
    xm = x_ref[...] + (jnp.dot(attn_ref[...], wo_ref[...],
                               preferred_element_type=jnp.float32) + bo_ref[...])
    xm_ref[...] = xm
    h2 = _ln(xm, g_ref[...], b_ref[...])
    h2_ref[...] = h2
    logit = jnp.dot(h2, wr_ref[...], preferred_element_type=jnp.float32) + br_ref[...]
    lane = jax.lax.broadcasted_iota(jnp.int32, logit.shape, 1)
    m1 = jnp.max(logit, axis=1, keepdims=True)
    i1 = jnp.min(jnp.where(logit == m1, lane, RL - 1), axis=1, keepdims=True)
    l2 = jnp.where(lane == i1, NEG, logit)
    m2 = jnp.max(l2, axis=1, keepdims=True)
    i2 = jnp.min(jnp.where(l2 == m2, lane, RL - 1), axis=1, keepdims=True)
    pe = jnp.exp(logit - m1)
    probs_ref[...] = pe / jnp.sum(pe, axis=1, keepdims=True)
    e2 = jnp.exp(m2 - m1)
    denom = 1.0 + e2
    w1 = 1.0 / denom
    w2 = e2 / denom
    idx_ref[...] = jnp.where(lane == 0, i1, i2)
    w_ref[...] = jnp.where(lane == 0, w1, w2)


def _proj_router(attn, xs, Wo, bo, g, b, Wr_pad, br_pad):
    row = pl.BlockSpec((BT, D), lambda i: (i, 0))
    rrow = pl.BlockSpec((BT, RL), lambda i: (i, 0))
    return pl.pallas_call(
        _proj_router_body,
        grid=(S // BT,),
        in_specs=[row, row,
                  pl.BlockSpec((D, D), lambda i: (0, 0)),
                  pl.BlockSpec((1, D), lambda i: (0, 0)),
                  pl.BlockSpec((1, D), lambda i: (0, 0)),
                  pl.BlockSpec((1, D), lambda i: (0, 0)),
                  pl.BlockSpec((D, RL), lambda i: (0, 0)),
                  pl.BlockSpec((1, RL), lambda i: (0, 0))],
        out_specs=[row, row, rrow, rrow, rrow],
        out_shape=[jax.ShapeDtypeStruct((S, D), jnp.float32),
                   jax.ShapeDtypeStruct((S, D), jnp.float32),
                   jax.ShapeDtypeStruct((S, RL), jnp.float32),
                   jax.ShapeDtypeStruct((S, RL), jnp.int32),
                   jax.ShapeDtypeStruct((S, RL), jnp.float32)],
    )(attn, xs, Wo, bo, g, b, Wr_pad, br_pad)


def _sc_gather(data, idx):
    N, Dd = data.shape
    M = idx.shape[0]
    SUB = 8
    CH = Dd // SUB
    W = 128
    data8 = data.reshape(N * SUB, CH)
    idx8 = (idx[:, None] * SUB + jnp.arange(SUB, dtype=jnp.int32)[None, :]
            ).reshape(1, M * SUB)
    mesh = plsc.VectorSubcoreMesh(core_axis_name="c", subcore_axis_name="s")

    @functools.partial(
        pl.kernel,
        out_type=jax.ShapeDtypeStruct((M * SUB, CH), data.dtype),
        mesh=mesh)
    def gk(x_hbm, i_hbm, o_hbm):
        def body(i_vmem, o_vmem):
            pltpu.sync_copy(x_hbm.at[i_vmem.at[0]], o_vmem)

        pltpu.emit_pipeline(
            body,
            grid=(M * SUB // W,),
            in_specs=[pl.BlockSpec((1, W), index_map=lambda i: (0, i))],
            out_specs=[pl.BlockSpec((W, CH), index_map=lambda i: (i, 0))],
            core_axis_name=("c", "s"),
            dimension_semantics=(pltpu.PARALLEL,),
        )(i_hbm, o_hbm)

    return gk(data8, idx8).reshape(M, Dd)


def _gate_up_body(te_ref, hs_ref, wg_ref, bg_ref, wu_ref, bu_ref, a_ref):
    hs = hs_ref[...]
    g = jnp.dot(hs, wg_ref[0], preferred_element_type=jnp.float32) + bg_ref[0]
    u = jnp.dot(hs, wu_ref[0], preferred_element_type=jnp.float32) + bu_ref[0]
    a_ref[...] = g * jax.lax.logistic(g) * u


def _gate_up(hs, Wg, bg3, Wu, bu3, te):
    grid_spec = pltpu.PrefetchScalarGridSpec(
        num_scalar_prefetch=1,
        grid=(F // BF, NT),
        in_specs=[
            pl.BlockSpec((TM, D), lambda f, t, te: (t, 0)),
            pl.BlockSpec((1, D, BF), lambda f, t, te: (te[t], 0, f)),
            pl.BlockSpec((1, 1, BF), lambda f, t, te: (te[t], 0, f)),
            pl.BlockSpec((1, D, BF), lambda f, t, te: (te[t], 0, f)),
            pl.BlockSpec((1, 1, BF), lambda f, t, te: (te[t], 0, f)),
        ],
        out_specs=pl.BlockSpec((TM, BF), lambda f, t, te: (t, f)),
    )
    return pl.pallas_call(
        _gate_up_body,
        grid_spec=grid_spec,
        out_shape=jax.ShapeDtypeStruct((PAD_P, F), jnp.float32),
    )(te, hs, Wg, bg3, Wu, bu3)


def _down_body(te_ref, a_ref, wd_ref, bd_ref, eo_ref):
    eo_ref[...] = jnp.dot(a_ref[...], wd_ref[0],
                          preferred_element_type=jnp.float32) + bd_ref[0]


def _down(a, Wd, bd3, te):
    grid_spec = pltpu.PrefetchScalarGridSpec(
        num_scalar_prefetch=1,
        grid=(NT,),
        in_specs=[
            pl.BlockSpec((TM, F), lambda t, te: (t, 0)),
            pl.BlockSpec((1, F, D), lambda t, te: (te[t], 0, 0)),
            pl.BlockSpec((1, 1, D), lambda t, te: (te[t], 0, 0)),
        ],
        out_specs=pl.BlockSpec((TM, D), lambda t, te: (t, 0)),
    )
    return pl.pallas_call(
        _down_body,
        grid_spec=grid_spec,
        out_shape=jax.ShapeDtypeStruct((PAD_P, D), jnp.float32),
    )(te, a, Wd, bd3)


def _combine_body(xm_ref, r2_ref, w_ref, o_ref):
    w1 = w_ref[:, 0:1]
    w2 = w_ref[:, 1:2]
    o_ref[...] = xm_ref[...] + w1 * r2_ref[:, :D] + w2 * r2_ref[:, D:]


def _combine(xm, r2, w):
    return pl.pallas_call(
        _combine_body,
        grid=(S // BT,),
        in_specs=[pl.BlockSpec((BT, D), lambda i: (i, 0)),
                  pl.BlockSpec((BT, 2 * D), lambda i: (i, 0)),
                  pl.BlockSpec((BT, RL), lambda i: (i, 0))],
        out_specs=pl.BlockSpec((BT, D), lambda i: (i, 0)),
        out_shape=jax.ShapeDtypeStruct((S, D), jnp.float32),
    )(xm, r2, w)


def _routing(idx2):
    ee = idx2.reshape(-1).astype(jnp.int32)
    order = jnp.argsort(ee, stable=True).astype(jnp.int32)
    counts = jnp.sum((ee[:, None] == jnp.arange(E)[None, :]).astype(jnp.int32), axis=0)
    padded = ((counts + TM - 1) // TM) * TM
    cum_pad = jnp.cumsum(padded)
    poff = cum_pad - padded
    cum_cnt = jnp.cumsum(counts)
    uoff = cum_cnt - counts
    e_sorted = ee[order]
    j = jnp.arange(K * S, dtype=jnp.int32)
    pos_sorted = (poff[e_sorted] + (j - uoff[e_sorted])).astype(jnp.int32)
    pos = jnp.zeros(K * S, jnp.int32).at[order].set(pos_sorted)
    src = jnp.zeros(PAD_P, jnp.int32).at[pos_sorted].set(order // K)
    te = jnp.searchsorted(cum_pad, jnp.arange(NT, dtype=jnp.int32) * TM,
                          side='right').astype(jnp.int32)
    te = jnp.minimum(te, E - 1)
    return pos, src, te


def kernel(x, ln1_g, ln1_b, ln2_g, ln2_b, Wq, bq, Wk, bk, Wv, bv, Wo, bo,
           Wr, br, Wg, bg, Wu, bu, Wd, bd):
    xs = x[0]
    g1 = ln1_g.reshape(1, D)
    b1 = ln1_b.reshape(1, D)
    g2 = ln2_g.reshape(1, D)
    b2 = ln2_b.reshape(1, D)
    Wr_pad = jnp.pad(Wr, ((0, 0), (0, RL - E)))
    br_pad = jnp.pad(br, (0, RL - E), constant_values=NEG).reshape(1, RL)

    q, k, v = _ln_qkv(xs, g1, b1, Wq, bq.reshape(1, D), Wk, bk.reshape(1, D),
                      Wv, bv.reshape(1, D))
    q = q.reshape(S, H, DH).transpose(1, 0, 2)
    k = k.reshape(S, H, DH).transpose(1, 0, 2)
    v = v.reshape(S, H, DH).transpose(1, 0, 2)
    attn = _attn(q, k, v).transpose(1, 0, 2).reshape(S, D)
    xm, h2, probs_pad, idx_pad, w_pad = _proj_router(
        attn, xs, Wo, bo.reshape(1, D), g2, b2, Wr_pad, br_pad)

    idx2 = idx_pad[:, :K]
    pos, src, te = _routing(idx2)

    hs = _sc_gather(h2, src)
    a = _gate_up(hs, Wg, bg.reshape(E, 1, F), Wu, bu.reshape(E, 1, F), te)
    eo = _down(a, Wd, bd.reshape(E, 1, D), te)
    r = _sc_gather(eo, pos)
    out = _combine(xm, r.reshape(S, K * D), w_pad)

    return (out[None], probs_pad[:, :E][None], idx2[None])
